# baseline (device time: 24948 ns/iter reference)
import functools

import jax
import jax.numpy as jnp
from jax import lax
from jax.experimental import pallas as pl
from jax.experimental.pallas import tpu as pltpu

N_DEV = 4


def kernel(x):
    m_per, n = x.shape
    half = m_per // 2

    def body(x_ref, out_ref, xv_ref, send_sems, recv_sems, local_sem):
        my = lax.axis_index("i")
        left = (my - 1) % N_DEV
        right = (my + 1) % N_DEV
        diag = (my + 2) % N_DEV

        def rows_a(j):
            return pl.ds(j * m_per, half)

        def rows_b(j):
            return pl.ds(j * m_per + half, half)

        def copy(src, dst, sem, dev):
            return pltpu.make_async_remote_copy(
                src_ref=src,
                dst_ref=dst,
                send_sem=send_sems.at[sem],
                recv_sem=recv_sems.at[sem],
                device_id=(dev,),
                device_id_type=pl.DeviceIdType.MESH,
            )

        stage = pltpu.make_async_copy(x_ref, xv_ref, local_sem)
        stage.start()

        barrier_sem = pltpu.get_barrier_semaphore()
        for nbr in [left, right]:
            pl.semaphore_signal(
                barrier_sem, inc=1,
                device_id=(nbr,), device_id_type=pl.DeviceIdType.MESH,
            )
        pl.semaphore_wait(barrier_sem, 2)
        stage.wait()

        t0 = copy(xv_ref.at[pl.ds(0, half), :], out_ref.at[rows_a(my), :], 0, right)
        t3 = copy(xv_ref.at[pl.ds(half, half), :], out_ref.at[rows_b(my), :], 3, left)
        t1 = copy(xv_ref.at[pl.ds(half, half), :], out_ref.at[rows_b(my), :], 1, right)
        t2 = copy(xv_ref.at[pl.ds(0, half), :], out_ref.at[rows_a(my), :], 2, left)
        t0.start()
        t3.start()
        t1.start()
        t2.start()

        own = pltpu.make_async_copy(
            xv_ref, out_ref.at[pl.ds(my * m_per, m_per), :], local_sem
        )
        own.start()

        copy(x_ref.at[pl.ds(0, half), :], out_ref.at[rows_a(left), :], 0, right).wait_recv()
        t4 = copy(out_ref.at[rows_a(left), :], out_ref.at[rows_a(left), :], 4, right)
        t4.start()

        copy(x_ref.at[pl.ds(half, half), :], out_ref.at[rows_b(right), :], 3, left).wait_recv()
        t5 = copy(out_ref.at[rows_b(right), :], out_ref.at[rows_b(right), :], 5, left)
        t5.start()

        copy(x_ref.at[pl.ds(half, half), :], out_ref.at[rows_b(left), :], 1, left).wait_recv()
        copy(x_ref.at[pl.ds(0, half), :], out_ref.at[rows_a(right), :], 2, right).wait_recv()
        copy(x_ref.at[pl.ds(0, half), :], out_ref.at[rows_a(diag), :], 4, left).wait_recv()
        copy(x_ref.at[pl.ds(half, half), :], out_ref.at[rows_b(diag), :], 5, right).wait_recv()

        own.wait()
        for t in [t0, t1, t2, t3, t4, t5]:
            t.wait_send()

        @functools.partial(
            pl.run_scoped, second_barrier=pltpu.SemaphoreType.REGULAR
        )
        def _(second_barrier):
            for nbr in [left, right]:
                pl.semaphore_signal(
                    second_barrier, inc=1,
                    device_id=(nbr,), device_id_type=pl.DeviceIdType.MESH,
                )
            pl.semaphore_wait(second_barrier, 2)

    x = pltpu.with_memory_space_constraint(x, pltpu.MemorySpace.HBM)
    return pl.pallas_call(
        body,
        out_shape=jax.ShapeDtypeStruct((N_DEV * m_per, n), x.dtype),
        in_specs=[pl.BlockSpec(memory_space=pltpu.MemorySpace.HBM)],
        out_specs=pl.BlockSpec(memory_space=pltpu.MemorySpace.HBM),
        scratch_shapes=[
            pltpu.VMEM((m_per, n), x.dtype),
            pltpu.SemaphoreType.DMA((6,)),
            pltpu.SemaphoreType.DMA((6,)),
            pltpu.SemaphoreType.DMA,
        ],
        compiler_params=pltpu.CompilerParams(collective_id=0),
    )(x)


# device time: 24407 ns/iter; 1.0222x vs baseline; 1.0222x over previous
import functools

import jax
import jax.numpy as jnp
from jax import lax
from jax.experimental import pallas as pl
from jax.experimental.pallas import tpu as pltpu

N_DEV = 4


def kernel(x):
    m_per, n = x.shape
    half = m_per // 2

    def body(x_ref, out_ref, send_sems, recv_sems, local_sem):
        my = lax.axis_index("i")
        left = (my - 1) % N_DEV
        right = (my + 1) % N_DEV
        diag = (my + 2) % N_DEV

        def rows_a(j):
            return pl.ds(j * m_per, half)

        def rows_b(j):
            return pl.ds(j * m_per + half, half)

        def copy(src, dst, sem, dev):
            return pltpu.make_async_remote_copy(
                src_ref=src,
                dst_ref=dst,
                send_sem=send_sems.at[sem],
                recv_sem=recv_sems.at[sem],
                device_id=(dev,),
                device_id_type=pl.DeviceIdType.MESH,
            )

        barrier_sem = pltpu.get_barrier_semaphore()
        for nbr in [left, right]:
            pl.semaphore_signal(
                barrier_sem, inc=1,
                device_id=(nbr,), device_id_type=pl.DeviceIdType.MESH,
            )
        pl.semaphore_wait(barrier_sem, 2)

        t0 = copy(x_ref.at[pl.ds(0, half), :], out_ref.at[rows_a(my), :], 0, right)
        t3 = copy(x_ref.at[pl.ds(half, half), :], out_ref.at[rows_b(my), :], 3, left)
        t1 = copy(x_ref.at[pl.ds(half, half), :], out_ref.at[rows_b(my), :], 1, right)
        t2 = copy(x_ref.at[pl.ds(0, half), :], out_ref.at[rows_a(my), :], 2, left)
        t0.start()
        t3.start()
        t1.start()
        t2.start()

        own = pltpu.make_async_copy(
            x_ref, out_ref.at[pl.ds(my * m_per, m_per), :], local_sem
        )
        own.start()

        copy(x_ref.at[pl.ds(0, half), :], out_ref.at[rows_a(left), :], 0, right).wait_recv()
        t4 = copy(out_ref.at[rows_a(left), :], out_ref.at[rows_a(left), :], 4, right)
        t4.start()

        copy(x_ref.at[pl.ds(half, half), :], out_ref.at[rows_b(right), :], 3, left).wait_recv()
        t5 = copy(out_ref.at[rows_b(right), :], out_ref.at[rows_b(right), :], 5, left)
        t5.start()

        copy(x_ref.at[pl.ds(half, half), :], out_ref.at[rows_b(left), :], 1, left).wait_recv()
        copy(x_ref.at[pl.ds(0, half), :], out_ref.at[rows_a(right), :], 2, right).wait_recv()
        copy(x_ref.at[pl.ds(0, half), :], out_ref.at[rows_a(diag), :], 4, left).wait_recv()
        copy(x_ref.at[pl.ds(half, half), :], out_ref.at[rows_b(diag), :], 5, right).wait_recv()

        own.wait()
        for t in [t0, t1, t2, t3, t4, t5]:
            t.wait_send()

        @functools.partial(
            pl.run_scoped, second_barrier=pltpu.SemaphoreType.REGULAR
        )
        def _(second_barrier):
            for nbr in [left, right]:
                pl.semaphore_signal(
                    second_barrier, inc=1,
                    device_id=(nbr,), device_id_type=pl.DeviceIdType.MESH,
                )
            pl.semaphore_wait(second_barrier, 2)

    x = pltpu.with_memory_space_constraint(x, pltpu.MemorySpace.HBM)
    return pl.pallas_call(
        body,
        out_shape=jax.ShapeDtypeStruct((N_DEV * m_per, n), x.dtype),
        in_specs=[pl.BlockSpec(memory_space=pltpu.MemorySpace.HBM)],
        out_specs=pl.BlockSpec(memory_space=pltpu.MemorySpace.HBM),
        scratch_shapes=[
            pltpu.SemaphoreType.DMA((6,)),
            pltpu.SemaphoreType.DMA((6,)),
            pltpu.SemaphoreType.DMA,
        ],
        compiler_params=pltpu.CompilerParams(collective_id=0),
    )(x)


# device time: 24065 ns/iter; 1.0367x vs baseline; 1.0142x over previous
import functools

import jax
import jax.numpy as jnp
from jax import lax
from jax.experimental import pallas as pl
from jax.experimental.pallas import tpu as pltpu

N_DEV = 4


def kernel(x):
    m_per, n = x.shape
    half = m_per // 2

    def body(x_ref, out_ref, send_sems, recv_sems, local_sem, rsem):
        my = lax.axis_index("i")
        left = (my - 1) % N_DEV
        right = (my + 1) % N_DEV
        diag = (my + 2) % N_DEV

        def rows_a(j):
            return pl.ds(j * m_per, half)

        def rows_b(j):
            return pl.ds(j * m_per + half, half)

        def copy(src, dst, sem, dev):
            return pltpu.make_async_remote_copy(
                src_ref=src,
                dst_ref=dst,
                send_sem=send_sems.at[sem],
                recv_sem=recv_sems.at[sem],
                device_id=(dev,),
                device_id_type=pl.DeviceIdType.MESH,
            )

        barrier_sem = pltpu.get_barrier_semaphore()
        pl.semaphore_signal(
            barrier_sem, inc=1,
            device_id=(right,), device_id_type=pl.DeviceIdType.MESH,
        )
        pl.semaphore_signal(
            rsem, inc=1,
            device_id=(left,), device_id_type=pl.DeviceIdType.MESH,
        )
        pl.semaphore_wait(barrier_sem, 1)
        pl.semaphore_wait(rsem, 1)

        t0 = copy(x_ref.at[pl.ds(0, half), :], out_ref.at[rows_a(my), :], 0, right)
        t3 = copy(x_ref.at[pl.ds(half, half), :], out_ref.at[rows_b(my), :], 3, left)
        t1 = copy(x_ref.at[pl.ds(half, half), :], out_ref.at[rows_b(my), :], 1, right)
        t2 = copy(x_ref.at[pl.ds(0, half), :], out_ref.at[rows_a(my), :], 2, left)
        t0.start()
        t3.start()
        t1.start()
        t2.start()

        own = pltpu.make_async_copy(
            x_ref, out_ref.at[pl.ds(my * m_per, m_per), :], local_sem
        )
        own.start()

        copy(x_ref.at[pl.ds(0, half), :], out_ref.at[rows_a(left), :], 0, right).wait_recv()
        t4 = copy(out_ref.at[rows_a(left), :], out_ref.at[rows_a(left), :], 4, right)
        t4.start()

        copy(x_ref.at[pl.ds(half, half), :], out_ref.at[rows_b(right), :], 3, left).wait_recv()
        t5 = copy(out_ref.at[rows_b(right), :], out_ref.at[rows_b(right), :], 5, left)
        t5.start()

        copy(x_ref.at[pl.ds(half, half), :], out_ref.at[rows_b(left), :], 1, left).wait_recv()
        copy(x_ref.at[pl.ds(0, half), :], out_ref.at[rows_a(right), :], 2, right).wait_recv()
        copy(x_ref.at[pl.ds(0, half), :], out_ref.at[rows_a(diag), :], 4, left).wait_recv()
        copy(x_ref.at[pl.ds(half, half), :], out_ref.at[rows_b(diag), :], 5, right).wait_recv()

        own.wait()
        for t in [t0, t1, t2, t3, t4, t5]:
            t.wait_send()

    x = pltpu.with_memory_space_constraint(x, pltpu.MemorySpace.HBM)
    return pl.pallas_call(
        body,
        out_shape=jax.ShapeDtypeStruct((N_DEV * m_per, n), x.dtype),
        in_specs=[pl.BlockSpec(memory_space=pltpu.MemorySpace.HBM)],
        out_specs=pl.BlockSpec(memory_space=pltpu.MemorySpace.HBM),
        scratch_shapes=[
            pltpu.SemaphoreType.DMA((6,)),
            pltpu.SemaphoreType.DMA((6,)),
            pltpu.SemaphoreType.DMA,
            pltpu.SemaphoreType.REGULAR,
        ],
        compiler_params=pltpu.CompilerParams(collective_id=0),
    )(x)
